# baseline (speedup 1.0000x reference)
import functools

import jax
import jax.numpy as jnp
from jax import lax
from jax.experimental import pallas as pl
from jax.experimental.pallas import tpu as pltpu
from jax.experimental.pallas import tpu_sc as plsc

V_MIN = -10.0
V_MAX = 10.0
N_ATOMS = 51
DISCOUNT = 0.99
DELTA = (V_MAX - V_MIN) / (N_ATOMS - 1)
INV_DELTA = 1.0 / DELTA

try:
    _info = plsc.get_sparse_core_info()
    NC, NS = _info.num_cores, _info.num_subcores
except Exception:
    NC, NS = 2, 16
NW = NC * NS

BLOCK = 128


def _zero_acc(accf, n16):
    zeros16 = jnp.zeros((16,), jnp.float32)

    def zbody(z, c):
        accf[pl.ds(z * 16, 16)] = zeros16
        return c

    lax.fori_loop(0, n16, zbody, 0, unroll=4)


def _body(rows_per, reward_h, probs_h, ndone_h, out_h,
          in0, in1, af0, af1, to0, to1, rw0, rw1, nd0, nd1,
          sin0, sin1, sout0, sout1):
    A = N_ATOMS
    B = BLOCK
    nblk = rows_per // B
    inbufs, accfs = (in0, in1), (af0, af1)
    touts = (to0, to1)
    rwbufs, ndbufs = (rw0, rw1), (nd0, nd1)
    sins, souts = (sin0, sin1), (sout0, sout1)
    wid = lax.axis_index("s") * NC + lax.axis_index("c")
    row_base = wid * rows_per
    lane = lax.iota(jnp.int32, 16)
    zeros16 = jnp.zeros((16,), jnp.float32)

    def start_in(b):
        r0 = row_base + b * B
        p = b % 2
        return [
            pltpu.async_copy(probs_h.at[pl.ds(r0, B)], inbufs[p], sins[p]),
            pltpu.async_copy(reward_h.at[pl.ds(r0, B)], rwbufs[p], sins[p]),
            pltpu.async_copy(ndone_h.at[pl.ds(r0, B)], ndbufs[p], sins[p]),
        ]

    pend_in = {0: start_in(0)}
    pend_out = {}
    _zero_acc(af0, (B * A) // 16)
    _zero_acc(af1, (B * A) // 16)
    for b in range(nblk):
        p = b % 2
        inb, accf, tout = inbufs[p], accfs[p], touts[p]
        rw, nd = rwbufs[p], ndbufs[p]
        for h in pend_in.pop(b):
            h.wait()
        if b + 1 < nblk:
            pend_in[b + 1] = start_in(b + 1)

        def group_body(g, c, inb=inb, accf=accf, rw=rw, nd=nd):
            s = g * 16
            rows = s + lane
            rowbase51 = rows * A
            rvec = rw[pl.ds(s, 16)]
            cvec = DISCOUNT * nd[pl.ds(s, 16)]
            base = (rvec - V_MIN) * INV_DELTA + (V_MIN * INV_DELTA) * cvec

            def j_body(j, carry, inb=inb, accf=accf, rows=rows,
                       cvec=cvec, rowbase51=rowbase51):
                b_unc, jsplat = carry
                bq = jnp.minimum(jnp.maximum(b_unc, 0.0), float(A - 1))
                li = bq.astype(jnp.int32)
                frac = bq - li.astype(jnp.float32)
                pvals = plsc.load_gather(inb, [rows, jsplat])
                wu = frac * pvals
                wl = pvals - wu
                idx_l = rowbase51 + li
                idx_u = idx_l + 1
                plsc.addupdate_scatter(accf, [idx_l], wl)
                plsc.addupdate_scatter(accf, [idx_u], wu)
                return (b_unc + cvec, jsplat + 1)

            lax.fori_loop(0, A, j_body,
                          (base, jnp.zeros((16,), jnp.int32)), unroll=6)
            return c

        lax.fori_loop(0, B // 16, group_body, 0)
        if b - 2 in pend_out:
            pend_out.pop(b - 2).wait()

        def repack_body(r, c, accf=accf, tout=tout):
            f0 = r * A
            v0 = accf[pl.ds(f0, 16)]
            v1 = accf[pl.ds(f0 + 16, 16)]
            v2 = accf[pl.ds(f0 + 32, 16)]
            v3 = accf[pl.ds(f0 + (A - 16), 16)]
            tout[r, pl.ds(0, 16)] = v0
            tout[r, pl.ds(16, 16)] = v1
            tout[r, pl.ds(32, 16)] = v2
            tout[r, pl.ds(A - 16, 16)] = v3
            accf[pl.ds(f0, 16)] = zeros16
            accf[pl.ds(f0 + 16, 16)] = zeros16
            accf[pl.ds(f0 + 32, 16)] = zeros16
            accf[pl.ds(f0 + (A - 16), 16)] = zeros16
            return c

        lax.fori_loop(0, B, repack_body, 0, unroll=2)
        r0 = row_base + b * B
        pend_out[b] = pltpu.async_copy(tout, out_h.at[pl.ds(r0, B)], souts[p])
    for h in pend_out.values():
        h.wait()


def kernel(reward, probs, not_done):
    bs, A = probs.shape
    assert A == N_ATOMS
    rows_per = bs // NW
    mesh = plsc.VectorSubcoreMesh(
        core_axis_name="c", subcore_axis_name="s",
        num_cores=NC, num_subcores=NS)
    run = functools.partial(
        pl.kernel,
        out_type=jax.ShapeDtypeStruct((bs, A), jnp.float32),
        mesh=mesh,
        compiler_params=pltpu.CompilerParams(
            needs_layout_passes=False, use_tc_tiling_on_sc=True,
            disable_bounds_checks=True),
        scratch_types=[
            pltpu.VMEM((BLOCK, A), jnp.float32),
            pltpu.VMEM((BLOCK, A), jnp.float32),
            pltpu.VMEM((BLOCK * A + 16,), jnp.float32),
            pltpu.VMEM((BLOCK * A + 16,), jnp.float32),
            pltpu.VMEM((BLOCK, A), jnp.float32),
            pltpu.VMEM((BLOCK, A), jnp.float32),
            pltpu.VMEM((BLOCK,), jnp.float32),
            pltpu.VMEM((BLOCK,), jnp.float32),
            pltpu.VMEM((BLOCK,), jnp.float32),
            pltpu.VMEM((BLOCK,), jnp.float32),
            pltpu.SemaphoreType.DMA,
            pltpu.SemaphoreType.DMA,
            pltpu.SemaphoreType.DMA,
            pltpu.SemaphoreType.DMA,
        ],
    )(functools.partial(_body, rows_per))
    return run(reward.reshape(-1), probs, not_done.reshape(-1))


# speedup vs baseline: 1.1739x; 1.1739x over previous
import functools

import jax
import jax.numpy as jnp
from jax import lax
from jax.experimental import pallas as pl
from jax.experimental.pallas import tpu as pltpu
from jax.experimental.pallas import tpu_sc as plsc

V_MIN = -10.0
V_MAX = 10.0
N_ATOMS = 51
DISCOUNT = 0.99
DELTA = (V_MAX - V_MIN) / (N_ATOMS - 1)
INV_DELTA = 1.0 / DELTA

try:
    _info = plsc.get_sparse_core_info()
    NC, NS = _info.num_cores, _info.num_subcores
except Exception:
    NC, NS = 2, 16
NW = NC * NS

BLOCK = 128


def _zero_acc(accf, n16):
    zeros16 = jnp.zeros((16,), jnp.float32)

    def zbody(z, c):
        accf[pl.ds(z * 16, 16)] = zeros16
        return c

    lax.fori_loop(0, n16, zbody, 0, unroll=4)


def _body(rows_per, reward_h, probs_h, ndone_h, out_h,
          in0, in1, if0, if1, af0, af1, to0, to1, rw0, rw1, nd0, nd1,
          sin0, sin1, sout0, sout1):
    A = N_ATOMS
    B = BLOCK
    nblk = rows_per // B
    inbufs, accfs = (in0, in1), (af0, af1)
    influts = (if0, if1)
    touts = (to0, to1)
    rwbufs, ndbufs = (rw0, rw1), (nd0, nd1)
    sins, souts = (sin0, sin1), (sout0, sout1)
    wid = lax.axis_index("s") * NC + lax.axis_index("c")
    row_base = wid * rows_per
    lane = lax.iota(jnp.int32, 16)
    zeros16 = jnp.zeros((16,), jnp.float32)

    def start_in(b):
        r0 = row_base + b * B
        p = b % 2
        return [
            pltpu.async_copy(probs_h.at[pl.ds(r0, B)], inbufs[p], sins[p]),
            pltpu.async_copy(reward_h.at[pl.ds(r0, B)], rwbufs[p], sins[p]),
            pltpu.async_copy(ndone_h.at[pl.ds(r0, B)], ndbufs[p], sins[p]),
        ]

    pend_in = {0: start_in(0)}
    pend_out = {}
    _zero_acc(af0, (B * A) // 16)
    _zero_acc(af1, (B * A) // 16)
    for b in range(nblk):
        p = b % 2
        inb, accf, tout = inbufs[p], accfs[p], touts[p]
        inf = influts[p]
        rw, nd = rwbufs[p], ndbufs[p]
        for h in pend_in.pop(b):
            h.wait()
        if b + 1 < nblk:
            pend_in[b + 1] = start_in(b + 1)

        def rin_body(r, c, inb=inb, inf=inf):
            f0 = r * A
            inf[pl.ds(f0, 16)] = inb[r, pl.ds(0, 16)]
            inf[pl.ds(f0 + 16, 16)] = inb[r, pl.ds(16, 16)]
            inf[pl.ds(f0 + 32, 16)] = inb[r, pl.ds(32, 16)]
            inf[pl.ds(f0 + (A - 16), 16)] = inb[r, pl.ds(A - 16, 16)]
            return c

        lax.fori_loop(0, B, rin_body, 0, unroll=2)

        def group_body(g, c, inf=inf, accf=accf, rw=rw, nd=nd):
            s = g * 16
            rows = s + lane
            rowbase51 = rows * A
            rvec = rw[pl.ds(s, 16)]
            cvec = DISCOUNT * nd[pl.ds(s, 16)]
            base = (rvec - V_MIN) * INV_DELTA + (V_MIN * INV_DELTA) * cvec

            def j_body(j, carry, inf=inf, accf=accf,
                       cvec=cvec, rowbase51=rowbase51):
                b_unc, gidx = carry
                bq = jnp.minimum(jnp.maximum(b_unc, 0.0), float(A - 1))
                li = bq.astype(jnp.int32)
                frac = bq - li.astype(jnp.float32)
                pvals = plsc.load_gather(inf, [gidx])
                wu = frac * pvals
                wl = pvals - wu
                idx_l = rowbase51 + li
                idx_u = idx_l + 1
                plsc.addupdate_scatter(accf, [idx_l], wl)
                plsc.addupdate_scatter(accf, [idx_u], wu)
                return (b_unc + cvec, gidx + 1)

            lax.fori_loop(0, A, j_body, (base, rowbase51), unroll=3)
            return c

        lax.fori_loop(0, B // 16, group_body, 0)
        if b - 2 in pend_out:
            pend_out.pop(b - 2).wait()

        def repack_body(r, c, accf=accf, tout=tout):
            f0 = r * A
            v0 = accf[pl.ds(f0, 16)]
            v1 = accf[pl.ds(f0 + 16, 16)]
            v2 = accf[pl.ds(f0 + 32, 16)]
            v3 = accf[pl.ds(f0 + (A - 16), 16)]
            tout[r, pl.ds(0, 16)] = v0
            tout[r, pl.ds(16, 16)] = v1
            tout[r, pl.ds(32, 16)] = v2
            tout[r, pl.ds(A - 16, 16)] = v3
            accf[pl.ds(f0, 16)] = zeros16
            accf[pl.ds(f0 + 16, 16)] = zeros16
            accf[pl.ds(f0 + 32, 16)] = zeros16
            accf[pl.ds(f0 + (A - 16), 16)] = zeros16
            return c

        lax.fori_loop(0, B, repack_body, 0, unroll=2)
        r0 = row_base + b * B
        pend_out[b] = pltpu.async_copy(tout, out_h.at[pl.ds(r0, B)], souts[p])
    for h in pend_out.values():
        h.wait()


def kernel(reward, probs, not_done):
    bs, A = probs.shape
    assert A == N_ATOMS
    rows_per = bs // NW
    mesh = plsc.VectorSubcoreMesh(
        core_axis_name="c", subcore_axis_name="s",
        num_cores=NC, num_subcores=NS)
    run = functools.partial(
        pl.kernel,
        out_type=jax.ShapeDtypeStruct((bs, A), jnp.float32),
        mesh=mesh,
        compiler_params=pltpu.CompilerParams(
            needs_layout_passes=False, use_tc_tiling_on_sc=True,
            disable_bounds_checks=True),
        scratch_types=[
            pltpu.VMEM((BLOCK, A), jnp.float32),
            pltpu.VMEM((BLOCK, A), jnp.float32),
            pltpu.VMEM((BLOCK * A,), jnp.float32),
            pltpu.VMEM((BLOCK * A,), jnp.float32),
            pltpu.VMEM((BLOCK * A + 16,), jnp.float32),
            pltpu.VMEM((BLOCK * A + 16,), jnp.float32),
            pltpu.VMEM((BLOCK, A), jnp.float32),
            pltpu.VMEM((BLOCK, A), jnp.float32),
            pltpu.VMEM((BLOCK,), jnp.float32),
            pltpu.VMEM((BLOCK,), jnp.float32),
            pltpu.VMEM((BLOCK,), jnp.float32),
            pltpu.VMEM((BLOCK,), jnp.float32),
            pltpu.SemaphoreType.DMA,
            pltpu.SemaphoreType.DMA,
            pltpu.SemaphoreType.DMA,
            pltpu.SemaphoreType.DMA,
        ],
    )(functools.partial(_body, rows_per))
    return run(reward.reshape(-1), probs, not_done.reshape(-1))
